# baseline (speedup 1.0000x reference)
import jax
import jax.numpy as jnp
from jax import lax
from jax.experimental import pallas as pl
from jax.experimental.pallas import tpu as pltpu
from jax.experimental.pallas import tpu_sc as plsc

NC = 2
NS = 16
L = 16
NW = NC * NS

B = 16384
D = 128
CH = 128
BPW = B // NW
NCHUNK = BPW // CH
GROUPS = CH // L
NVEC = D // L


def _body(user_h, item_h, uf_h, if_h, ubg_h, ibg_h, out_h,
          uidx, iidx, urows0, urows1, irows0, irows1, ubv, ibv,
          accm, outv, sem0, sem1, semb):
    cid = lax.axis_index("c")
    sid = lax.axis_index("s")
    wid = sid * NC + cid
    base = wid * BPW

    for ch in range(NCHUNK):
        pltpu.sync_copy(user_h.at[pl.ds(base + ch * CH, CH)], uidx.at[ch])
        pltpu.sync_copy(item_h.at[pl.ds(base + ch * CH, CH)], iidx.at[ch])

    urows = (urows0, urows1)
    irows = (irows0, irows1)
    sems = (sem0, sem1)

    def fire(ch):
        b = ch % 2
        s = sems[b]
        return (
            pltpu.async_copy(uf_h.at[uidx.at[ch]], urows[b], s),
            pltpu.async_copy(if_h.at[iidx.at[ch]], irows[b], s),
        )

    def fire_bias(ch):
        return (
            pltpu.async_copy(ubg_h.at[pl.ds(base + ch * CH, CH)], ubv, semb),
            pltpu.async_copy(ibg_h.at[pl.ds(base + ch * CH, CH)], ibv, semb),
        )

    iota = lax.iota(jnp.int32, L)
    zeros = jnp.zeros((L,), jnp.int32)

    def compute(ch):
        b = ch % 2
        u = urows[b]
        v = irows[b]

        def group_body(g, _):
            rowbase = g * L
            for r in range(L):
                row = rowbase + r
                acc = u[row, pl.ds(0, L)] * v[row, pl.ds(0, L)]
                for j in range(1, NVEC):
                    acc = acc + u[row, pl.ds(j * L, L)] * v[row, pl.ds(j * L, L)]
                plsc.store_scatter(accm, [iota, jnp.full((L,), r, jnp.int32)], acc)
            tot = accm[0, :]
            for j in range(1, L):
                tot = tot + accm[j, :]
            rows = rowbase + iota
            tot = tot + plsc.load_gather(ubv, [rows, zeros])
            tot = tot + plsc.load_gather(ibv, [rows, zeros])
            outv[pl.ds(ch * CH + rowbase, L)] = tot
            return 0

        lax.fori_loop(0, GROUPS, group_body, 0)

    pending = {0: fire(0)}
    pb = fire_bias(0)
    for ch in range(NCHUNK):
        if ch + 1 < NCHUNK:
            pending[ch + 1] = fire(ch + 1)
        for d in pending.pop(ch):
            d.wait()
        for d in pb:
            d.wait()
        compute(ch)
        if ch + 1 < NCHUNK:
            pb = fire_bias(ch + 1)

    pltpu.sync_copy(outv, out_h.at[pl.ds(base, BPW)])


@jax.jit
def _run(user, item, user_factors, item_factors, ubg, ibg):
    mesh = plsc.VectorSubcoreMesh(core_axis_name="c", subcore_axis_name="s")
    f = pl.kernel(
        _body,
        out_type=jax.ShapeDtypeStruct((B,), jnp.float32),
        mesh=mesh,
        compiler_params=pltpu.CompilerParams(needs_layout_passes=False),
        scratch_types=[
            pltpu.VMEM((NCHUNK, CH), jnp.int32),
            pltpu.VMEM((NCHUNK, CH), jnp.int32),
            pltpu.VMEM((CH, D), jnp.float32),
            pltpu.VMEM((CH, D), jnp.float32),
            pltpu.VMEM((CH, D), jnp.float32),
            pltpu.VMEM((CH, D), jnp.float32),
            pltpu.VMEM((CH, 1), jnp.float32),
            pltpu.VMEM((CH, 1), jnp.float32),
            pltpu.VMEM((L, L), jnp.float32),
            pltpu.VMEM((BPW,), jnp.float32),
            pltpu.SemaphoreType.DMA,
            pltpu.SemaphoreType.DMA,
            pltpu.SemaphoreType.DMA,
        ],
    )
    return f(user, item, user_factors, item_factors, ubg, ibg)


def kernel(user, item, user_factors, item_factors, user_biases, item_biases):
    ubg = jnp.take(user_biases, user, axis=0)
    ibg = jnp.take(item_biases, item, axis=0)
    return _run(user, item, user_factors, item_factors, ubg, ibg)


# speedup vs baseline: 1.3341x; 1.3341x over previous
import jax
import jax.numpy as jnp
from jax import lax
from jax.experimental import pallas as pl
from jax.experimental.pallas import tpu as pltpu
from jax.experimental.pallas import tpu_sc as plsc

NC = 2
NS = 16
L = 16
NW = NC * NS

B = 16384
D = 128
CH = 128
BPW = B // NW
NCHUNK = BPW // CH
GROUPS = CH // L
NVEC = D // L


def _body(user_h, item_h, uf_h, if_h, ubg_h, ibg_h, out_h,
          uidx, iidx, urows0, urows1, irows0, irows1, ubv, ibv,
          accm, outv, sem0, sem1, semb):
    cid = lax.axis_index("c")
    sid = lax.axis_index("s")
    wid = sid * NC + cid
    base = wid * BPW

    for ch in range(NCHUNK):
        pltpu.sync_copy(user_h.at[pl.ds(base + ch * CH, CH)], uidx.at[ch])
        pltpu.sync_copy(item_h.at[pl.ds(base + ch * CH, CH)], iidx.at[ch])

    urows = (urows0, urows1)
    irows = (irows0, irows1)
    sems = (sem0, sem1)

    def fire(ch):
        b = ch % 2
        s = sems[b]
        return (
            pltpu.async_copy(uf_h.at[uidx.at[ch]], urows[b], s),
            pltpu.async_copy(if_h.at[iidx.at[ch]], irows[b], s),
        )

    def fire_bias(ch):
        return (
            pltpu.async_copy(ubg_h.at[uidx.at[ch]], ubv.at[ch], semb),
            pltpu.async_copy(ibg_h.at[iidx.at[ch]], ibv.at[ch], semb),
        )

    iota = lax.iota(jnp.int32, L)

    def compute(ch):
        b = ch % 2
        u = urows[b]
        v = irows[b]

        def group_body(g, _):
            rowbase = g * L
            for r in range(L):
                row = rowbase + r
                acc = u[row, pl.ds(0, L)] * v[row, pl.ds(0, L)]
                for j in range(1, NVEC):
                    acc = acc + u[row, pl.ds(j * L, L)] * v[row, pl.ds(j * L, L)]
                plsc.store_scatter(accm, [iota, jnp.full((L,), r, jnp.int32)], acc)
            tot = accm[0, :]
            for j in range(1, L):
                tot = tot + accm[j, :]
            tot = tot + ubv[ch, pl.ds(rowbase, L)] + ibv[ch, pl.ds(rowbase, L)]
            outv[pl.ds(ch * CH + rowbase, L)] = tot
            return 0

        lax.fori_loop(0, GROUPS, group_body, 0)

    pending = {0: fire(0) + fire_bias(0)}
    for ch in range(NCHUNK):
        if ch + 1 < NCHUNK:
            pending[ch + 1] = fire(ch + 1) + fire_bias(ch + 1)
        for d in pending.pop(ch):
            d.wait()
        compute(ch)

    pltpu.sync_copy(outv, out_h.at[pl.ds(base, BPW)])


@jax.jit
def _run(user, item, user_factors, item_factors, ubg, ibg):
    mesh = plsc.VectorSubcoreMesh(core_axis_name="c", subcore_axis_name="s")
    f = pl.kernel(
        _body,
        out_type=jax.ShapeDtypeStruct((B,), jnp.float32),
        mesh=mesh,
        compiler_params=pltpu.CompilerParams(needs_layout_passes=False),
        scratch_types=[
            pltpu.VMEM((NCHUNK, CH), jnp.int32),
            pltpu.VMEM((NCHUNK, CH), jnp.int32),
            pltpu.VMEM((CH, D), jnp.float32),
            pltpu.VMEM((CH, D), jnp.float32),
            pltpu.VMEM((CH, D), jnp.float32),
            pltpu.VMEM((CH, D), jnp.float32),
            pltpu.VMEM((NCHUNK, CH), jnp.float32),
            pltpu.VMEM((NCHUNK, CH), jnp.float32),
            pltpu.VMEM((L, L), jnp.float32),
            pltpu.VMEM((BPW,), jnp.float32),
            pltpu.SemaphoreType.DMA,
            pltpu.SemaphoreType.DMA,
            pltpu.SemaphoreType.DMA,
        ],
    )
    return f(user, item, user_factors, item_factors, ubg, ibg)


def kernel(user, item, user_factors, item_factors, user_biases, item_biases):
    ubt, ibt = lax.optimization_barrier((user_biases.T, item_biases.T))
    ubg = ubt.reshape(-1)
    ibg = ibt.reshape(-1)
    return _run(user, item, user_factors, item_factors, ubg, ibg)


# speedup vs baseline: 1.6313x; 1.2228x over previous
import jax
import jax.numpy as jnp
from jax import lax
from jax.experimental import pallas as pl
from jax.experimental.pallas import tpu as pltpu
from jax.experimental.pallas import tpu_sc as plsc

NC = 2
NS = 16
L = 16
NW = NC * NS

B = 16384
D = 128
CH = 128
BPW = B // NW
NCHUNK = BPW // CH
GROUPS = CH // L
NVEC = D // L


def _dots_body(user_h, item_h, uf_h, if_h, out_h,
               uidx, iidx, urows0, urows1, irows0, irows1,
               accm, outv, sem0, sem1):
    cid = lax.axis_index("c")
    sid = lax.axis_index("s")
    wid = sid * NC + cid
    base = wid * BPW

    for ch in range(NCHUNK):
        pltpu.sync_copy(user_h.at[pl.ds(base + ch * CH, CH)], uidx.at[ch])
        pltpu.sync_copy(item_h.at[pl.ds(base + ch * CH, CH)], iidx.at[ch])

    urows = (urows0, urows1)
    irows = (irows0, irows1)
    sems = (sem0, sem1)

    def fire(ch):
        b = ch % 2
        s = sems[b]
        return (
            pltpu.async_copy(uf_h.at[uidx.at[ch]], urows[b], s),
            pltpu.async_copy(if_h.at[iidx.at[ch]], irows[b], s),
        )

    iota = lax.iota(jnp.int32, L)

    def compute(ch):
        b = ch % 2
        u = urows[b]
        v = irows[b]

        def group_body(g, _):
            rowbase = g * L
            for r in range(L):
                row = rowbase + r
                acc = u[row, pl.ds(0, L)] * v[row, pl.ds(0, L)]
                for j in range(1, NVEC):
                    acc = acc + u[row, pl.ds(j * L, L)] * v[row, pl.ds(j * L, L)]
                plsc.store_scatter(accm, [iota, jnp.full((L,), r, jnp.int32)], acc)
            tot = accm[0, :]
            for j in range(1, L):
                tot = tot + accm[j, :]
            outv[pl.ds(ch * CH + rowbase, L)] = tot
            return 0

        lax.fori_loop(0, GROUPS, group_body, 0)

    pending = {0: fire(0)}
    for ch in range(NCHUNK):
        if ch + 1 < NCHUNK:
            pending[ch + 1] = fire(ch + 1)
        for d in pending.pop(ch):
            d.wait()
        compute(ch)

    pltpu.sync_copy(outv, out_h.at[pl.ds(base, BPW)])


def _bias_body(user_h, item_h, ub_h, ib_h, dots_h, out_h,
               uidx, iidx, ubv, ibv, dv, outv, sem):
    cid = lax.axis_index("c")
    sid = lax.axis_index("s")
    wid = sid * NC + cid
    base = wid * BPW

    for ch in range(NCHUNK):
        pltpu.sync_copy(user_h.at[pl.ds(base + ch * CH, CH)], uidx.at[ch])
        pltpu.sync_copy(item_h.at[pl.ds(base + ch * CH, CH)], iidx.at[ch])

    descs = []
    for ch in range(NCHUNK):
        descs.append(pltpu.async_copy(ub_h.at[uidx.at[ch]], ubv.at[ch], sem))
        descs.append(pltpu.async_copy(ib_h.at[iidx.at[ch]], ibv.at[ch], sem))
    pltpu.sync_copy(dots_h.at[pl.ds(base, BPW)], dv)
    for d in descs:
        d.wait()

    for ch in range(NCHUNK):
        for g in range(GROUPS):
            rowbase = g * L
            pos = ch * CH + rowbase
            outv[pl.ds(pos, L)] = (
                dv[pl.ds(pos, L)]
                + ubv[ch, pl.ds(rowbase, L)]
                + ibv[ch, pl.ds(rowbase, L)]
            )

    pltpu.sync_copy(outv, out_h.at[pl.ds(base, BPW)])


@jax.jit
def _run(user, item, user_factors, item_factors, user_biases, item_biases):
    mesh = plsc.VectorSubcoreMesh(core_axis_name="c", subcore_axis_name="s")
    params = pltpu.CompilerParams(needs_layout_passes=False)

    dots_fn = pl.kernel(
        _dots_body,
        out_type=jax.ShapeDtypeStruct((B,), jnp.float32),
        mesh=mesh,
        compiler_params=params,
        scratch_types=[
            pltpu.VMEM((NCHUNK, CH), jnp.int32),
            pltpu.VMEM((NCHUNK, CH), jnp.int32),
            pltpu.VMEM((CH, D), jnp.float32),
            pltpu.VMEM((CH, D), jnp.float32),
            pltpu.VMEM((CH, D), jnp.float32),
            pltpu.VMEM((CH, D), jnp.float32),
            pltpu.VMEM((L, L), jnp.float32),
            pltpu.VMEM((BPW,), jnp.float32),
            pltpu.SemaphoreType.DMA,
            pltpu.SemaphoreType.DMA,
        ],
    )

    bias_fn = pl.kernel(
        _bias_body,
        out_type=jax.ShapeDtypeStruct((B,), jnp.float32),
        mesh=mesh,
        compiler_params=params,
        scratch_types=[
            pltpu.VMEM((NCHUNK, CH), jnp.int32),
            pltpu.VMEM((NCHUNK, CH), jnp.int32),
            pltpu.VMEM((NCHUNK, CH), jnp.float32),
            pltpu.VMEM((NCHUNK, CH), jnp.float32),
            pltpu.VMEM((BPW,), jnp.float32),
            pltpu.VMEM((BPW,), jnp.float32),
            pltpu.SemaphoreType.DMA,
        ],
    )

    ubt, ibt = lax.optimization_barrier((user_biases.T, item_biases.T))
    ubg = ubt.reshape(-1)
    ibg = ibt.reshape(-1)

    dots = dots_fn(user, item, user_factors, item_factors)
    return bias_fn(user, item, ubg, ibg, dots)


def kernel(user, item, user_factors, item_factors, user_biases, item_biases):
    return _run(user, item, user_factors, item_factors, user_biases, item_biases)


# speedup vs baseline: 1.6952x; 1.0392x over previous
import jax
import jax.numpy as jnp
from jax import lax
from jax.experimental import pallas as pl
from jax.experimental.pallas import tpu as pltpu
from jax.experimental.pallas import tpu_sc as plsc

NC = 2
NS = 16
L = 16
NW = NC * NS

B = 16384
D = 128
CH = 128
BPW = B // NW
NCHUNK = BPW // CH
GROUPS = CH // L
NVEC = D // L


def _dots_body(user_h, item_h, uf_h, if_h, ib_h, out_h,
               uidx, iidx, urows0, urows1, irows0, irows1, ibv,
               accm, outv, sem0, sem1, semi):
    cid = lax.axis_index("c")
    sid = lax.axis_index("s")
    wid = sid * NC + cid
    base = wid * BPW

    idx_descs = []
    for ch in range(NCHUNK):
        idx_descs.append(pltpu.async_copy(
            user_h.at[pl.ds(base + ch * CH, CH)], uidx.at[ch], semi))
        idx_descs.append(pltpu.async_copy(
            item_h.at[pl.ds(base + ch * CH, CH)], iidx.at[ch], semi))
    for d in idx_descs:
        d.wait()

    urows = (urows0, urows1)
    irows = (irows0, irows1)
    sems = (sem0, sem1)

    def fire(ch):
        b = ch % 2
        s = sems[b]
        return (
            pltpu.async_copy(uf_h.at[uidx.at[ch]], urows[b], s),
            pltpu.async_copy(if_h.at[iidx.at[ch]], irows[b], s),
        )

    ib_descs = [
        pltpu.async_copy(ib_h.at[iidx.at[ch]], ibv.at[ch], semi)
        for ch in range(NCHUNK)
    ]

    iota = lax.iota(jnp.int32, L)

    def compute(ch):
        b = ch % 2
        u = urows[b]
        v = irows[b]

        def group_body(g, _):
            rowbase = g * L
            for r in range(L):
                row = rowbase + r
                acc = u[row, pl.ds(0, L)] * v[row, pl.ds(0, L)]
                for j in range(1, NVEC):
                    acc = acc + u[row, pl.ds(j * L, L)] * v[row, pl.ds(j * L, L)]
                plsc.store_scatter(accm, [iota, jnp.full((L,), r, jnp.int32)], acc)
            tot = accm[0, :]
            for j in range(1, L):
                tot = tot + accm[j, :]
            tot = tot + ibv[ch, pl.ds(rowbase, L)]
            outv[pl.ds(ch * CH + rowbase, L)] = tot
            return 0

        lax.fori_loop(0, GROUPS, group_body, 0)

    pending = {0: fire(0)}
    for ch in range(NCHUNK):
        if ch + 1 < NCHUNK:
            pending[ch + 1] = fire(ch + 1)
        for d in pending.pop(ch):
            d.wait()
        if ch == 0:
            for d in ib_descs:
                d.wait()
        compute(ch)

    pltpu.sync_copy(outv, out_h.at[pl.ds(base, BPW)])


def _bias_body(user_h, ub_h, dots_h, out_h, uidx, ubv, dv, outv, sem):
    cid = lax.axis_index("c")
    sid = lax.axis_index("s")
    wid = sid * NC + cid
    base = wid * BPW

    idx_descs = [
        pltpu.async_copy(user_h.at[pl.ds(base + ch * CH, CH)], uidx.at[ch], sem)
        for ch in range(NCHUNK)
    ]
    dd = pltpu.async_copy(dots_h.at[pl.ds(base, BPW)], dv, sem)
    for d in idx_descs:
        d.wait()
    descs = [
        pltpu.async_copy(ub_h.at[uidx.at[ch]], ubv.at[ch], sem)
        for ch in range(NCHUNK)
    ]
    dd.wait()
    for d in descs:
        d.wait()

    for ch in range(NCHUNK):
        for g in range(GROUPS):
            rowbase = g * L
            pos = ch * CH + rowbase
            outv[pl.ds(pos, L)] = dv[pl.ds(pos, L)] + ubv[ch, pl.ds(rowbase, L)]

    pltpu.sync_copy(outv, out_h.at[pl.ds(base, BPW)])


@jax.jit
def _run(user, item, user_factors, item_factors, user_biases, item_biases):
    mesh = plsc.VectorSubcoreMesh(core_axis_name="c", subcore_axis_name="s")
    params = pltpu.CompilerParams(needs_layout_passes=False)

    dots_fn = pl.kernel(
        _dots_body,
        out_type=jax.ShapeDtypeStruct((B,), jnp.float32),
        mesh=mesh,
        compiler_params=params,
        scratch_types=[
            pltpu.VMEM((NCHUNK, CH), jnp.int32),
            pltpu.VMEM((NCHUNK, CH), jnp.int32),
            pltpu.VMEM((CH, D), jnp.float32),
            pltpu.VMEM((CH, D), jnp.float32),
            pltpu.VMEM((CH, D), jnp.float32),
            pltpu.VMEM((CH, D), jnp.float32),
            pltpu.VMEM((NCHUNK, CH), jnp.float32),
            pltpu.VMEM((L, L), jnp.float32),
            pltpu.VMEM((BPW,), jnp.float32),
            pltpu.SemaphoreType.DMA,
            pltpu.SemaphoreType.DMA,
            pltpu.SemaphoreType.DMA,
        ],
    )

    bias_fn = pl.kernel(
        _bias_body,
        out_type=jax.ShapeDtypeStruct((B,), jnp.float32),
        mesh=mesh,
        compiler_params=params,
        scratch_types=[
            pltpu.VMEM((NCHUNK, CH), jnp.int32),
            pltpu.VMEM((NCHUNK, CH), jnp.float32),
            pltpu.VMEM((BPW,), jnp.float32),
            pltpu.VMEM((BPW,), jnp.float32),
            pltpu.SemaphoreType.DMA,
        ],
    )

    ubt, ibt = lax.optimization_barrier((user_biases.T, item_biases.T))
    ubg = ubt.reshape(-1)
    ibg = ibt.reshape(-1)

    dots = dots_fn(user, item, user_factors, item_factors, ibg)
    return bias_fn(user, ubg, dots)


def kernel(user, item, user_factors, item_factors, user_biases, item_biases):
    return _run(user, item, user_factors, item_factors, user_biases, item_biases)


# speedup vs baseline: 1.6964x; 1.0007x over previous
import jax
import jax.numpy as jnp
from jax import lax
from jax.experimental import pallas as pl
from jax.experimental.pallas import tpu as pltpu
from jax.experimental.pallas import tpu_sc as plsc

NC = 2
NS = 16
L = 16
NW = NC * NS

B = 16384
D = 128
CH = 128
BPW = B // NW
NCHUNK = BPW // CH
GROUPS = CH // L
NVEC = D // L


def _dots_body(user_h, item_h, uf_h, if_h, ib_h, out_h,
               uidx, iidx, urows0, urows1, irows0, irows1, ibv,
               accm, outv, sem0, sem1, semi):
    cid = lax.axis_index("c")
    sid = lax.axis_index("s")
    wid = sid * NC + cid
    base = wid * BPW

    idx_descs = []
    for ch in range(NCHUNK):
        idx_descs.append(pltpu.async_copy(
            user_h.at[pl.ds(base + ch * CH, CH)], uidx.at[ch], semi))
        idx_descs.append(pltpu.async_copy(
            item_h.at[pl.ds(base + ch * CH, CH)], iidx.at[ch], semi))
    for d in idx_descs:
        d.wait()

    urows = (urows0, urows1)
    irows = (irows0, irows1)
    sems = (sem0, sem1)

    def fire(ch):
        b = ch % 2
        s = sems[b]
        return (
            pltpu.async_copy(uf_h.at[uidx.at[ch]], urows[b], s),
            pltpu.async_copy(if_h.at[iidx.at[ch]], irows[b], s),
        )

    ib_descs = [
        pltpu.async_copy(ib_h.at[iidx.at[ch]], ibv.at[ch], semi)
        for ch in range(NCHUNK)
    ]

    iota = lax.iota(jnp.int32, L)

    def compute(ch):
        b = ch % 2
        u = urows[b]
        v = irows[b]

        def group_body(g, _):
            rowbase = g * L
            for r in range(L):
                row = rowbase + r
                acc = u[row, pl.ds(0, L)] * v[row, pl.ds(0, L)]
                for j in range(1, NVEC):
                    acc = acc + u[row, pl.ds(j * L, L)] * v[row, pl.ds(j * L, L)]
                plsc.store_scatter(accm, [iota, jnp.full((L,), r, jnp.int32)], acc)
            tot = accm[0, :]
            for j in range(1, L):
                tot = tot + accm[j, :]
            tot = tot + ibv[ch, pl.ds(rowbase, L)]
            outv[pl.ds(ch * CH + rowbase, L)] = tot
            return 0

        lax.fori_loop(0, GROUPS, group_body, 0)

    pending = {0: fire(0)}
    for ch in range(NCHUNK):
        if ch + 1 < NCHUNK:
            pending[ch + 1] = fire(ch + 1)
        for d in pending.pop(ch):
            d.wait()
        if ch == 0:
            for d in ib_descs:
                d.wait()
        compute(ch)

    pltpu.sync_copy(outv, out_h.at[pl.ds(base, BPW)])


def _bias_body(user_h, ub_h, dots_h, out_h, uidx, ubv, dv, outv, sem):
    cid = lax.axis_index("c")
    sid = lax.axis_index("s")
    wid = sid * NC + cid
    base = wid * BPW

    idx_descs = [
        pltpu.async_copy(user_h.at[pl.ds(base + ch * CH, CH)], uidx.at[ch], sem)
        for ch in range(NCHUNK)
    ]
    dd = pltpu.async_copy(dots_h.at[pl.ds(base, BPW)], dv, sem)
    for d in idx_descs:
        d.wait()
    descs = [
        pltpu.async_copy(ub_h.at[uidx.at[ch]], ubv.at[ch], sem)
        for ch in range(NCHUNK)
    ]
    dd.wait()
    for d in descs:
        d.wait()

    for ch in range(NCHUNK):
        for g in range(GROUPS):
            rowbase = g * L
            pos = ch * CH + rowbase
            outv[pl.ds(pos, L)] = dv[pl.ds(pos, L)] + ubv[ch, pl.ds(rowbase, L)]

    pltpu.sync_copy(outv, out_h.at[pl.ds(base, BPW)])


@jax.jit
def _run(user, item, user_factors, item_factors, user_biases, item_biases):
    mesh = plsc.VectorSubcoreMesh(core_axis_name="c", subcore_axis_name="s")
    params = pltpu.CompilerParams(needs_layout_passes=False)

    dots_fn = pl.kernel(
        _dots_body,
        out_type=jax.ShapeDtypeStruct((B,), jnp.float32),
        mesh=mesh,
        compiler_params=params,
        scratch_types=[
            pltpu.VMEM((NCHUNK, CH), jnp.int32),
            pltpu.VMEM((NCHUNK, CH), jnp.int32),
            pltpu.VMEM((CH, D), jnp.float32),
            pltpu.VMEM((CH, D), jnp.float32),
            pltpu.VMEM((CH, D), jnp.float32),
            pltpu.VMEM((CH, D), jnp.float32),
            pltpu.VMEM((NCHUNK, CH), jnp.float32),
            pltpu.VMEM((L, L), jnp.float32),
            pltpu.VMEM((BPW,), jnp.float32),
            pltpu.SemaphoreType.DMA,
            pltpu.SemaphoreType.DMA,
            pltpu.SemaphoreType.DMA,
        ],
    )

    bias_fn = pl.kernel(
        _bias_body,
        out_type=jax.ShapeDtypeStruct((B,), jnp.float32),
        mesh=mesh,
        compiler_params=params,
        scratch_types=[
            pltpu.VMEM((NCHUNK, CH), jnp.int32),
            pltpu.VMEM((NCHUNK, CH), jnp.float32),
            pltpu.VMEM((BPW,), jnp.float32),
            pltpu.VMEM((BPW,), jnp.float32),
            pltpu.SemaphoreType.DMA,
        ],
    )

    ubt = lax.optimization_barrier(user_biases.T)
    ubg = ubt.reshape(-1)
    ibt = lax.optimization_barrier(item_biases.T)
    ibg = ibt.reshape(-1)

    dots = dots_fn(user, item, user_factors, item_factors, ibg)
    return bias_fn(user, ubg, dots)


def kernel(user, item, user_factors, item_factors, user_biases, item_biases):
    return _run(user, item, user_factors, item_factors, user_biases, item_biases)
